# baseline (speedup 1.0000x reference)
import functools
import jax
import jax.numpy as jnp
from jax import lax
from jax.experimental import pallas as pl
from jax.experimental.pallas import tpu as pltpu
from jax.experimental.pallas import tpu_sc as plsc

S, D, F, E, K = 2048, 1024, 4096, 8, 2
A = S * K
BLK = 128
P = A + E * BLK
NB = P // BLK
PT = P + BLK
DUMP_T = P
SLOTS_N = A + 64
DUMP_S = A
NTILE = 32
LANES = 16

_mesh = plsc.VectorSubcoreMesh(core_axis_name="c", subcore_axis_name="s")


def _gelu(v):
    return 0.5 * v * (1.0 + jax.lax.erf(v * 0.7071067811865476))


def _router_body(x_ref, rw_ref, rb_ref, eid_ref, w_ref):
    logits = jax.lax.dot_general(
        x_ref[...], rw_ref[...], (((1,), (1,)), ((), ())),
        preferred_element_type=jnp.float32) + rb_ref[...][None, :]
    ids = jax.lax.broadcasted_iota(jnp.int32, logits.shape, 1)
    m1 = jnp.max(logits, axis=1, keepdims=True)
    i1 = jnp.min(jnp.where(logits == m1, ids, E), axis=1, keepdims=True)
    sel1 = ids == i1
    l1 = jnp.sum(jnp.where(sel1, logits, 0.0), axis=1, keepdims=True)
    lm = jnp.where(sel1, -jnp.inf, logits)
    m2 = jnp.max(lm, axis=1, keepdims=True)
    i2 = jnp.min(jnp.where(lm == m2, ids, E), axis=1, keepdims=True)
    l2 = jnp.sum(jnp.where(ids == i2, lm, 0.0), axis=1, keepdims=True)
    w1 = jax.nn.sigmoid(l1 - l2)
    eid_ref[...] = jnp.concatenate([i1, i2], axis=1)
    w_ref[...] = jnp.concatenate([w1, 1.0 - w1], axis=1)


def _router(x2):
    return pl.pallas_call(
        _router_body,
        in_specs=[
            pl.BlockSpec((S, D), lambda: (0, 0)),
            pl.BlockSpec((E, D), lambda: (0, 0)),
            pl.BlockSpec((E,), lambda: (0,)),
        ],
        out_specs=[
            pl.BlockSpec((S, K), lambda: (0, 0)),
            pl.BlockSpec((S, K), lambda: (0, 0)),
        ],
        out_shape=[
            jax.ShapeDtypeStruct((S, K), jnp.int32),
            jax.ShapeDtypeStruct((S, K), jnp.float32),
        ],
    )


NV = A // LANES
ZCH = PT // 8
NROW = A // BLK


def _dispatch_body(eid_hbm, w_hbm, stok_hbm, sw_hbm, slots_hbm, cnt_hbm,
                   eid_v, w_v, cnt_v, zbuf, zbuff, tidx_v, ttok_v, tw_v,
                   sidx_v, sval_v, sem):
    cid = lax.axis_index("c")
    sid = lax.axis_index("s")
    lane = lax.iota(jnp.int32, LANES)

    @pl.when((cid == 0) & (sid >= 8))
    def _zero_fill():
        z = jnp.zeros((LANES,), jnp.int32)
        zf = jnp.zeros((LANES,), jnp.float32)
        for i in range(ZCH // LANES):
            zbuf[pl.ds(i * LANES, LANES)] = z
            zbuff[pl.ds(i * LANES, LANES)] = zf
        base = (sid - 8) * ZCH
        pltpu.sync_copy(zbuf, stok_hbm.at[pl.ds(base, ZCH)])
        pltpu.sync_copy(zbuff, sw_hbm.at[pl.ds(base, ZCH)])

    @pl.when((cid == 0) & (sid < 8))
    def _count():
        pltpu.sync_copy(eid_hbm, eid_v)
        pltpu.sync_copy(w_hbm, w_v)

        def cbody(i, acc):
            v = eid_v[pl.ds(pl.multiple_of(i * LANES, LANES), LANES)]
            for e in range(E):
                pc = jnp.sum(jnp.where(v == e, 1, 0))
                acc = acc + jnp.where(lane == e, pc, 0)
            return acc
        counts = lax.fori_loop(0, NV, cbody, jnp.zeros((LANES,), jnp.int32))
        cnt_v[...] = counts

        @pl.when(sid == 0)
        def _():
            pltpu.sync_copy(cnt_v, cnt_hbm)

    plsc.subcore_barrier()

    @pl.when((cid == 0) & (sid < 8))
    def _rank_scatter():
        myexp = sid
        counts = cnt_v[...]
        cap = ((counts + (BLK - 1)) // BLK) * BLK
        mybase = jnp.sum(jnp.where(lane < myexp, cap, 0))

        run = jnp.int32(0)
        for r in range(NROW):
            def sbody(i2, run):
                i = r * 8 + i2
                off = pl.multiple_of(i * LANES, LANES)
                v = eid_v[pl.ds(off, LANES)]
                wv = w_v[pl.ds(off, LANES)]
                jv = i * LANES + lane
                m = v == myexp
                mi = jnp.where(m, 1, 0)
                pref = plsc.cumsum(mi)
                rank = mybase + run + pref - 1
                o2 = pl.multiple_of(i2 * LANES, LANES)
                tidx_v.at[r][pl.ds(o2, LANES)] = jnp.where(m, rank, DUMP_T)
                ttok_v.at[r][pl.ds(o2, LANES)] = jv >> 1
                tw_v.at[r][pl.ds(o2, LANES)] = wv
                sidx_v.at[r][pl.ds(o2, LANES)] = jnp.where(
                    m, (jv & 1) * S + (jv >> 1), DUMP_S)
                sval_v.at[r][pl.ds(o2, LANES)] = rank
                return run + jnp.sum(mi)
            run = lax.fori_loop(0, 8, sbody, run)

        copies = []
        for r in range(NROW):
            copies.append(pltpu.async_copy(
                ttok_v.at[r], stok_hbm.at[tidx_v.at[r]], sem))
            copies.append(pltpu.async_copy(
                tw_v.at[r], sw_hbm.at[tidx_v.at[r]], sem))
            copies.append(pltpu.async_copy(
                sval_v.at[r], slots_hbm.at[sidx_v.at[r]], sem))
        for c in copies:
            c.wait()


def _dispatch(eid_flat, w_flat):
    fn = pl.kernel(
        _dispatch_body,
        out_type=(
            jax.ShapeDtypeStruct((PT,), jnp.int32),
            jax.ShapeDtypeStruct((PT,), jnp.float32),
            jax.ShapeDtypeStruct((SLOTS_N,), jnp.int32),
            jax.ShapeDtypeStruct((LANES,), jnp.int32),
        ),
        mesh=_mesh,
        compiler_params=pltpu.CompilerParams(needs_layout_passes=False),
        scratch_types=[
            pltpu.VMEM((A,), jnp.int32),
            pltpu.VMEM((A,), jnp.float32),
            pltpu.VMEM((LANES,), jnp.int32),
            pltpu.VMEM((ZCH,), jnp.int32),
            pltpu.VMEM((ZCH,), jnp.float32),
            pltpu.VMEM((NROW, BLK), jnp.int32),
            pltpu.VMEM((NROW, BLK), jnp.int32),
            pltpu.VMEM((NROW, BLK), jnp.float32),
            pltpu.VMEM((NROW, BLK), jnp.int32),
            pltpu.VMEM((NROW, BLK), jnp.int32),
            pltpu.SemaphoreType.DMA,
        ],
    )
    return fn(eid_flat, w_flat)


RPT = P // NTILE
GCH = 80


def _gather_body(x_hbm, stok_hbm, xs_hbm, idxb, rows, sem):
    cid = lax.axis_index("c")
    sid = lax.axis_index("s")
    wid = sid * 2 + cid
    base = wid * RPT
    for c in range(RPT // GCH):
        pltpu.sync_copy(stok_hbm.at[pl.ds(base + c * GCH, GCH)], idxb.at[c])
        pltpu.async_copy(x_hbm.at[idxb.at[c]], rows, sem).wait()
        pltpu.sync_copy(rows, xs_hbm.at[pl.ds(base + c * GCH, GCH), :])


def _gather_rows(x2, stok):
    fn = pl.kernel(
        _gather_body,
        out_type=jax.ShapeDtypeStruct((P, D), jnp.float32),
        mesh=_mesh,
        compiler_params=pltpu.CompilerParams(needs_layout_passes=False),
        scratch_types=[
            pltpu.VMEM((RPT // GCH, GCH), jnp.int32),
            pltpu.VMEM((GCH, D), jnp.float32),
            pltpu.SemaphoreType.DMA,
        ],
    )
    return fn(x2, stok)


def _block_expert(b, cnt_ref):
    cum = 0
    e = 0
    for ee in range(E):
        cum = cum + (cnt_ref[ee] + (BLK - 1)) // BLK
        e = e + jnp.where(b >= cum, 1, 0)
    return jnp.minimum(e, E - 1)


def _fc1_body(cnt_ref, xs_ref, w1_ref, b1_ref, g_ref, h_ref):
    h = jax.lax.dot_general(
        xs_ref[...], w1_ref[0], (((1,), (1,)), ((), ())),
        preferred_element_type=jnp.float32) + b1_ref[0]
    h_ref[...] = _gelu(h * g_ref[0])


def _fc1(counts, xs, fc1_w, fc1_b, gate):
    grid_spec = pltpu.PrefetchScalarGridSpec(
        num_scalar_prefetch=1,
        grid=(NB,),
        in_specs=[
            pl.BlockSpec((BLK, D), lambda b, c: (b, 0)),
            pl.BlockSpec((1, F, D), lambda b, c: (_block_expert(b, c), 0, 0)),
            pl.BlockSpec((1, 1, F), lambda b, c: (_block_expert(b, c), 0, 0)),
            pl.BlockSpec((1, 1, F), lambda b, c: (_block_expert(b, c), 0, 0)),
        ],
        out_specs=pl.BlockSpec((BLK, F), lambda b, c: (b, 0)),
    )
    return pl.pallas_call(
        _fc1_body,
        grid_spec=grid_spec,
        out_shape=jax.ShapeDtypeStruct((P, F), jnp.float32),
        compiler_params=pltpu.CompilerParams(vmem_limit_bytes=60000 * 1024),
    )(counts, xs, fc1_w, fc1_b.reshape(E, 1, F), gate.reshape(E, 1, F))


def _fc2_body(cnt_ref, h_ref, w2_ref, b2_ref, sw_ref, ys_ref):
    ys = jax.lax.dot_general(
        h_ref[...], w2_ref[0], (((1,), (1,)), ((), ())),
        preferred_element_type=jnp.float32) + b2_ref[0]
    ys_ref[...] = ys * sw_ref[0]


def _fc2(counts, h, fc2_w, fc2_b, sorted_w):
    grid_spec = pltpu.PrefetchScalarGridSpec(
        num_scalar_prefetch=1,
        grid=(NB,),
        in_specs=[
            pl.BlockSpec((BLK, F), lambda b, c: (b, 0)),
            pl.BlockSpec((1, D, F), lambda b, c: (_block_expert(b, c), 0, 0)),
            pl.BlockSpec((1, 1, D), lambda b, c: (_block_expert(b, c), 0, 0)),
            pl.BlockSpec((1, BLK, 1), lambda b, c: (b, 0, 0)),
        ],
        out_specs=pl.BlockSpec((BLK, D), lambda b, c: (b, 0)),
    )
    return pl.pallas_call(
        _fc2_body,
        grid_spec=grid_spec,
        out_shape=jax.ShapeDtypeStruct((P, D), jnp.float32),
        compiler_params=pltpu.CompilerParams(vmem_limit_bytes=60000 * 1024),
    )(counts, h, fc2_w, fc2_b.reshape(E, 1, D),
      sorted_w[:P].reshape(NB, BLK, 1))


FCH = 512
NF = F // FCH


def _shared_body(x_ref, w1_ref, b1_ref, g_ref, w2_ref, b2_ref, sw_ref,
                 out_ref, acc_ref):
    f = pl.program_id(0)
    sig = jax.nn.sigmoid(sw_ref[0])

    @pl.when(f == 0)
    def _():
        acc_ref[...] = jnp.broadcast_to(sig * b2_ref[...][None, :], (S, D))

    h = jax.lax.dot_general(
        x_ref[...], w1_ref[...], (((1,), (1,)), ((), ())),
        preferred_element_type=jnp.float32) + b1_ref[...][None, :]
    h = _gelu(h * g_ref[...][None, :])
    acc_ref[...] += sig * jax.lax.dot_general(
        h, w2_ref[...], (((1,), (1,)), ((), ())),
        preferred_element_type=jnp.float32)

    @pl.when(f == NF - 1)
    def _():
        out_ref[...] = acc_ref[...]


def _shared(x2, sfc1_w, sfc1_b, sgate, sfc2_w, sfc2_b, shared_weight):
    return pl.pallas_call(
        _shared_body,
        grid=(NF,),
        in_specs=[
            pl.BlockSpec((S, D), lambda f: (0, 0)),
            pl.BlockSpec((FCH, D), lambda f: (f, 0)),
            pl.BlockSpec((FCH,), lambda f: (f,)),
            pl.BlockSpec((FCH,), lambda f: (f,)),
            pl.BlockSpec((D, FCH), lambda f: (0, f)),
            pl.BlockSpec((D,), lambda f: (0,)),
            pl.BlockSpec(memory_space=pltpu.SMEM),
        ],
        out_specs=pl.BlockSpec((S, D), lambda f: (0, 0)),
        out_shape=jax.ShapeDtypeStruct((S, D), jnp.float32),
        scratch_shapes=[pltpu.VMEM((S, D), jnp.float32)],
        compiler_params=pltpu.CompilerParams(vmem_limit_bytes=60000 * 1024),
    )(x2, sfc1_w, sfc1_b, sgate, sfc2_w, sfc2_b, shared_weight.reshape(1))


TPT = S // NTILE


def _combine_body(slots_ref, so_ref, ys_ref, out_ref, acc_ref):
    b = pl.program_id(0)

    @pl.when(b == 0)
    def _():
        acc_ref[...] = so_ref[...]

    pids = b * BLK + jax.lax.broadcasted_iota(jnp.int32, (1, BLK), 1)
    sel = ((slots_ref[0] == pids) | (slots_ref[1] == pids)).astype(jnp.float32)
    acc_ref[...] += jax.lax.dot_general(
        sel, ys_ref[...], (((1,), (0,)), ((), ())),
        preferred_element_type=jnp.float32)

    @pl.when(b == NB - 1)
    def _():
        out_ref[...] = acc_ref[...]


def _combine(so, ys, slots):
    return pl.pallas_call(
        _combine_body,
        grid=(NB,),
        in_specs=[
            pl.BlockSpec((K, S, 1), lambda b: (0, 0, 0)),
            pl.BlockSpec((S, D), lambda b: (0, 0)),
            pl.BlockSpec((BLK, D), lambda b: (b, 0)),
        ],
        out_specs=pl.BlockSpec((S, D), lambda b: (0, 0)),
        out_shape=jax.ShapeDtypeStruct((S, D), jnp.float32),
        scratch_shapes=[pltpu.VMEM((S, D), jnp.float32)],
        compiler_params=pltpu.CompilerParams(vmem_limit_bytes=60000 * 1024),
    )(slots[:K * S].reshape(K, S, 1), so, ys)


def kernel(x, router_w, router_b, fc1_w, fc1_b, gate, fc2_w, fc2_b,
           sfc1_w, sfc1_b, sgate, sfc2_w, sfc2_b, shared_weight):
    x2 = x.reshape(S, D)
    eid, w = _router(x2)(x2, router_w, router_b)
    stok, sw, slots, counts = _dispatch(eid.reshape(A), w.reshape(A))
    xs = _gather_rows(x2, stok)
    h = _fc1(counts, xs, fc1_w, fc1_b, gate)
    ys = _fc2(counts, h, fc2_w, fc2_b, sw)
    so = _shared(x2, sfc1_w, sfc1_b, sgate, sfc2_w, sfc2_b, shared_weight)
    out = _combine(so, ys, slots)
    return out.reshape(1, S, D)
